# Initial kernel scaffold; baseline (speedup 1.0000x reference)
import jax
import jax.numpy as jnp
from jax.experimental import pallas as pl


def kernel(fts, adj, batch, W1, b1, W2, b2, W3, b3, Wl, bl, Wf, bf):
    raise NotImplementedError("write your pallas kernel here")


# speedup vs baseline: 6.6743x; 6.6743x over previous
import functools

import jax
import jax.numpy as jnp
from jax import lax
from jax.experimental import pallas as pl
from jax.experimental.pallas import tpu as pltpu
from jax.experimental.pallas import tpu_sc as plsc

N = 50000
E = 800000
IN = 9
H = 64
G = 512

NC = 2
NS = 16

NP = 53248
NROW = NP // 16
HALF = NP // 2
TPN = HALF // NS
TRN = TPN // 16
EP = 819200
EPT = EP // NS
CH = 128
NCHUNK = EPT // CH
DEGR = 53504
DUMP_DST = 53500
ACCR = HALF + NS
GR = 640
MAGIC = 0x5F3759DF

_mesh = functools.partial(
    plsc.VectorSubcoreMesh, core_axis_name="c", subcore_axis_name="s",
    num_cores=NC, num_subcores=NS)
_sc_params = pltpu.CompilerParams(
    needs_layout_passes=False, use_tc_tiling_on_sc=False)
f32 = jnp.float32
i32 = jnp.int32


def _prep_body(dst_hbm, batch_hbm, iota_hbm, zeros_hbm, ones_hbm,
               dinv_hbm, cnt_hbm,
               ebuf, onesb, cntp, bbuf, iv, draw, dinvb, deg_sh, cnt_sh):
    c = lax.axis_index("c")
    s = lax.axis_index("s")
    ones = jnp.full((16,), 1.0, f32)
    lane = jnp.arange(16, dtype=i32)
    zeros16 = jnp.zeros((16,), i32)

    pltpu.sync_copy(zeros_hbm.at[pl.ds(0, DEGR // NS)],
                    deg_sh.at[pl.ds(s * (DEGR // NS), DEGR // NS)])
    pltpu.sync_copy(zeros_hbm.at[pl.ds(0, GR // NS)],
                    cnt_sh.at[pl.ds(s * (GR // NS), GR // NS)])
    pltpu.sync_copy(zeros_hbm.at[pl.ds(0, GR)], cntp)
    pltpu.sync_copy(ones_hbm, onesb)
    pltpu.sync_copy(iota_hbm, iv)
    plsc.subcore_barrier()

    def deg_chunk(j, carry):
        base = s * EPT + j * CH
        pltpu.sync_copy(dst_hbm.at[pl.ds(base, CH)], ebuf.at[0])
        pltpu.sync_copy(onesb, deg_sh.at[ebuf.at[0]], add=True)
        return carry
    lax.fori_loop(0, NCHUNK, deg_chunk, 0)

    @pl.when(c == 0)
    def _():
        pltpu.sync_copy(batch_hbm.at[pl.ds(s * (NROW // NS), NROW // NS)],
                        bbuf)
        for r in range(NROW // NS):
            plsc.addupdate_scatter(cntp, [bbuf[r], lane], ones)

    plsc.subcore_barrier()

    @pl.when(c == 0)
    def _():
        for k in range(GR // 128):
            pltpu.sync_copy(cntp.at[pl.ds(k * 128, 128)],
                            cnt_sh.at[iv.at[k]], add=True)

    plsc.subcore_barrier()

    node_base = c * HALF + s * TPN
    pltpu.sync_copy(deg_sh.at[pl.ds(node_base, TPN)], draw)
    for r in range(TRN):
        x = plsc.load_gather(draw, [r * 16 + lane, zeros16]) + 1.0
        xi = plsc.bitcast(x, i32)
        y = plsc.bitcast(MAGIC - lax.shift_right_logical(xi, 1), f32)
        for _ in range(3):
            y = y * (1.5 - 0.5 * x * y * y)
        dinvb[r] = y
    pltpu.sync_copy(dinvb, dinv_hbm.at[pl.ds(c * (NROW // 2) + s * TRN, TRN)])

    @pl.when(jnp.logical_and(c == 0, s == 0))
    def _():
        pltpu.sync_copy(cnt_sh, cnt_hbm)


def _prep(dst_p, batch2d, iota8, zeros3344, ones128):
    return pl.kernel(
        _prep_body,
        out_type=(jax.ShapeDtypeStruct((NROW, 16), f32),
                  jax.ShapeDtypeStruct((GR, 16), f32)),
        mesh=_mesh(),
        compiler_params=_sc_params,
        scratch_types=[
            pltpu.VMEM((1, CH), i32),
            pltpu.VMEM((CH, 16), f32),
            pltpu.VMEM((GR, 16), f32),
            pltpu.VMEM((NROW // NS, 16), i32),
            pltpu.VMEM((8, 128), i32),
            pltpu.VMEM((TPN, 16), f32),
            pltpu.VMEM((TRN, 16), f32),
            pltpu.VMEM_SHARED((DEGR, 16), f32),
            pltpu.VMEM_SHARED((GR, 16), f32),
        ],
    )(dst_p, batch2d, iota8, zeros3344, ones128)


def _prop_body(u_hbm, src_hbm, dst_hbm, p_hbm, srcb, dstb, rowb, acc):
    c = lax.axis_index("c")
    s = lax.axis_index("s")
    node_base = c * HALF + s * TPN
    lo = c * HALF
    dump = HALF + s

    pltpu.sync_copy(u_hbm.at[pl.ds(node_base, TPN)],
                    acc.at[pl.ds(s * TPN, TPN)])
    plsc.subcore_barrier()

    def chunk(j, carry):
        base = s * EPT + j * CH
        pltpu.sync_copy(src_hbm.at[pl.ds(base, CH)], srcb)
        pltpu.sync_copy(dst_hbm.at[pl.ds(base, CH)], dstb.at[0])
        for g in range(CH // 16):
            d = dstb[0, pl.ds(g * 16, 16)]
            ld = d - lo
            m = jnp.logical_and(ld >= 0, ld < HALF)
            dstb[0, pl.ds(g * 16, 16)] = jnp.where(m, ld, dump)
        pltpu.sync_copy(u_hbm.at[srcb], rowb)
        pltpu.sync_copy(rowb, acc.at[dstb.at[0]], add=True)
        return carry
    lax.fori_loop(0, NCHUNK, chunk, 0)

    plsc.subcore_barrier()
    pltpu.sync_copy(acc.at[pl.ds(s * TPN, TPN)],
                    p_hbm.at[pl.ds(node_base, TPN)])


def _prop(u, src_p, dst_p):
    return pl.kernel(
        _prop_body,
        out_type=jax.ShapeDtypeStruct((NP, H), f32),
        mesh=_mesh(),
        compiler_params=pltpu.CompilerParams(use_tc_tiling_on_sc=False),
        scratch_types=[
            pltpu.VMEM((CH,), i32),
            pltpu.VMEM((1, CH), i32),
            pltpu.VMEM((CH, H), f32),
            pltpu.VMEM_SHARED((ACCR, H), f32),
        ],
    )(u, src_p, dst_p)


def _pool_body(z_hbm, batch_hbm, iota_hbm, zeros_hbm, psum_hbm,
               zb, bb, pp, iv, psh):
    c = lax.axis_index("c")
    s = lax.axis_index("s")
    row_base = c * (NROW // 2) + s * TRN
    lane = jnp.arange(16, dtype=i32)

    pltpu.sync_copy(zeros_hbm.at[pl.ds(0, GR)], pp)
    pltpu.sync_copy(zeros_hbm.at[pl.ds(0, GR // NS)],
                    psh.at[pl.ds(s * (GR // NS), GR // NS)])
    pltpu.sync_copy(iota_hbm, iv)
    pltpu.sync_copy(z_hbm.at[pl.ds(row_base, TRN)], zb)
    pltpu.sync_copy(batch_hbm.at[pl.ds(row_base, TRN)], bb)
    plsc.subcore_barrier()
    for r in range(TRN):
        plsc.addupdate_scatter(pp, [bb[r], lane], zb[r])
    for k in range(GR // 128):
        pltpu.sync_copy(pp.at[pl.ds(k * 128, 128)],
                        psh.at[iv.at[k]], add=True)
    plsc.subcore_barrier()
    @pl.when(s == 0)
    def _():
        pltpu.sync_copy(psh, psum_hbm.at[c])


def _pool(z2d, batch2d, iota8, zeros3344):
    return pl.kernel(
        _pool_body,
        out_type=jax.ShapeDtypeStruct((NC, GR, 16), f32),
        mesh=_mesh(),
        compiler_params=_sc_params,
        scratch_types=[
            pltpu.VMEM((TRN, 16), f32),
            pltpu.VMEM((TRN, 16), i32),
            pltpu.VMEM((GR, 16), f32),
            pltpu.VMEM((8, 128), i32),
            pltpu.VMEM_SHARED((GR, 16), f32),
        ],
    )(z2d, batch2d, iota8, zeros3344)


def _lrelu(x):
    return jnp.where(x >= 0, x, 0.01 * x)


def _tc_in_body(fts_ref, w_ref, dinv_ref, o_ref):
    xw = jnp.dot(fts_ref[...], w_ref[...], preferred_element_type=f32)
    o_ref[...] = dinv_ref[...] * xw


def _tc_mid_body(p_ref, dinv_ref, b_ref, w_ref, o_ref):
    h = _lrelu(dinv_ref[...] * p_ref[...] + b_ref[...])
    o_ref[...] = dinv_ref[...] * jnp.dot(h, w_ref[...],
                                         preferred_element_type=f32)


def _tc_out_body(p_ref, dinv_ref, b3_ref, wl_ref, bl_ref, wf_ref, o_ref):
    h3 = dinv_ref[...] * p_ref[...] + b3_ref[...]
    t = _lrelu(jnp.dot(h3, wl_ref[...], preferred_element_type=f32)
               + bl_ref[...])
    o_ref[...] = jnp.sum(t * wf_ref[...], axis=1, keepdims=True)


def _tc_fin_body(ps_ref, cnt_ref, bf_ref, o_ref):
    st = ps_ref[:GR, :] + ps_ref[GR:, :]
    pooled = jnp.sum(st, axis=1, keepdims=True)
    counts = jnp.sum(cnt_ref[...], axis=1, keepdims=True)
    mean = pooled[:G] / jnp.maximum(counts[:G], 1.0)
    o_ref[...] = mean + bf_ref[0]


_R = 512
_GRID = NP // _R


def _row_spec(w):
    return pl.BlockSpec((_R, w), lambda i: (i, 0))


def _full_spec(a, b):
    return pl.BlockSpec((a, b), lambda i: (0, 0))


def _vec_spec(n):
    return pl.BlockSpec((n,), lambda i: (0,))


def _tc_in(fts_p, w1p, dinv2):
    return pl.pallas_call(
        _tc_in_body,
        grid=(_GRID,),
        in_specs=[_row_spec(16), _full_spec(16, H), _row_spec(1)],
        out_specs=_row_spec(H),
        out_shape=jax.ShapeDtypeStruct((NP, H), f32),
    )(fts_p, w1p, dinv2)


def _tc_mid(p, dinv2, b, w):
    return pl.pallas_call(
        _tc_mid_body,
        grid=(_GRID,),
        in_specs=[_row_spec(H), _row_spec(1), _vec_spec(H),
                  _full_spec(H, H)],
        out_specs=_row_spec(H),
        out_shape=jax.ShapeDtypeStruct((NP, H), f32),
    )(p, dinv2, b, w)


def _tc_out(p, dinv2, b3, wl, bl, wfr):
    return pl.pallas_call(
        _tc_out_body,
        grid=(_GRID,),
        in_specs=[_row_spec(H), _row_spec(1), _vec_spec(H),
                  _full_spec(H, H), _vec_spec(H), _vec_spec(H)],
        out_specs=_row_spec(1),
        out_shape=jax.ShapeDtypeStruct((NP, 1), f32),
    )(p, dinv2, b3, wl, bl, wfr)


def _tc_fin(psumf, cntf, bf):
    return pl.pallas_call(
        _tc_fin_body,
        grid=(1,),
        in_specs=[_full_spec(NC * GR, 16), _full_spec(GR, 16),
                  _vec_spec(1)],
        out_specs=pl.BlockSpec((G, 1), lambda i: (0, 0)),
        out_shape=jax.ShapeDtypeStruct((G, 1), f32),
    )(psumf, cntf, bf)


def kernel(fts, adj, batch, W1, b1, W2, b2, W3, b3, Wl, bl, Wf, bf):
    fts_p = jnp.zeros((NP, 16), f32).at[:N, :IN].set(fts)
    w1p = jnp.zeros((16, H), f32).at[:IN, :].set(W1)
    src_p = jnp.zeros((EP,), i32).at[:E].set(adj[0])
    dst_p = jnp.full((EP,), DUMP_DST, i32).at[:E].set(adj[1])
    batch_p = jnp.full((NP,), G, i32).at[:N].set(batch)
    batch2d = batch_p.reshape(NROW, 16)
    iota8 = jnp.arange(8 * 128, dtype=i32).reshape(8, 128)
    zeros3344 = jnp.zeros((DEGR // NS, 16), f32)
    ones128 = jnp.ones((CH, 16), f32)

    dinv2d, cnt2d = _prep(dst_p, batch2d, iota8, zeros3344, ones128)
    dinv2 = dinv2d.reshape(NP, 1)

    u1 = _tc_in(fts_p, w1p, dinv2)
    p1 = _prop(u1, src_p, dst_p)
    u2 = _tc_mid(p1, dinv2, b1, W2)
    p2 = _prop(u2, src_p, dst_p)
    u3 = _tc_mid(p2, dinv2, b2, W3)
    p3 = _prop(u3, src_p, dst_p)
    z = _tc_out(p3, dinv2, b3, Wl, bl, Wf.reshape(H))
    psum = _pool(z.reshape(NROW, 16), batch2d, iota8, zeros3344)
    out = _tc_fin(psum.reshape(NC * GR, 16), cnt2d, bf)
    return out


# speedup vs baseline: 8.3864x; 1.2565x over previous
import functools

import jax
import jax.numpy as jnp
from jax import lax
from jax.experimental import pallas as pl
from jax.experimental.pallas import tpu as pltpu
from jax.experimental.pallas import tpu_sc as plsc

N = 50000
E = 800000
IN = 9
H = 64
G = 512

NC = 2
NS = 16

NP = 53248
NROW = NP // 16
HALF = NP // 2
TPN = HALF // NS
TRN = TPN // 16
EP = 819200
EPT = EP // NS
CH = 128
NCHUNK = EPT // CH
DEGR = 53504
DUMP_DST = 53500
ACCR = HALF + NS
GR = 640
MAGIC = 0x5F3759DF

_mesh = functools.partial(
    plsc.VectorSubcoreMesh, core_axis_name="c", subcore_axis_name="s",
    num_cores=NC, num_subcores=NS)
_sc_params = pltpu.CompilerParams(
    needs_layout_passes=False, use_tc_tiling_on_sc=False)
f32 = jnp.float32
i32 = jnp.int32


def _prep_body(dst_hbm, batch_hbm, iota_hbm, zeros_hbm, ones_hbm,
               dinv_hbm, cnt_hbm,
               ebuf, dsem, onesb, cntp, bbuf, iv, draw, dinvb, deg_sh,
               cnt_sh):
    c = lax.axis_index("c")
    s = lax.axis_index("s")
    ones = jnp.full((16,), 1.0, f32)
    lane = jnp.arange(16, dtype=i32)
    zeros16 = jnp.zeros((16,), i32)

    pltpu.sync_copy(zeros_hbm.at[pl.ds(0, DEGR // NS)],
                    deg_sh.at[pl.ds(s * (DEGR // NS), DEGR // NS)])
    pltpu.sync_copy(zeros_hbm.at[pl.ds(0, GR // NS)],
                    cnt_sh.at[pl.ds(s * (GR // NS), GR // NS)])
    pltpu.sync_copy(zeros_hbm.at[pl.ds(0, GR)], cntp)
    pltpu.sync_copy(ones_hbm, onesb)
    pltpu.sync_copy(iota_hbm, iv)
    plsc.subcore_barrier()

    def deg_chunk(j, carry):
        base = s * (NCHUNK // 8) * 8 + j * 8
        pltpu.sync_copy(dst_hbm.at[pl.ds(base, 8)], ebuf)
        dd = [pltpu.async_copy(onesb, deg_sh.at[ebuf.at[k]], dsem, add=True)
              for k in range(8)]
        for d_ in dd:
            d_.wait()
        return carry
    lax.fori_loop(0, NCHUNK // 8, deg_chunk, 0)

    @pl.when(c == 0)
    def _():
        pltpu.sync_copy(batch_hbm.at[pl.ds(s * (NROW // NS), NROW // NS)],
                        bbuf)
        for r in range(NROW // NS):
            plsc.addupdate_scatter(cntp, [bbuf[r], lane], ones)

    plsc.subcore_barrier()

    @pl.when(c == 0)
    def _():
        for k in range(GR // 128):
            pltpu.sync_copy(cntp.at[pl.ds(k * 128, 128)],
                            cnt_sh.at[iv.at[k]], add=True)

    plsc.subcore_barrier()

    node_base = c * HALF + s * TPN
    pltpu.sync_copy(deg_sh.at[pl.ds(node_base, TPN)], draw)
    for r in range(TRN):
        x = plsc.load_gather(draw, [r * 16 + lane, zeros16]) + 1.0
        xi = plsc.bitcast(x, i32)
        y = plsc.bitcast(MAGIC - lax.shift_right_logical(xi, 1), f32)
        for _ in range(3):
            y = y * (1.5 - 0.5 * x * y * y)
        dinvb[r] = y
    pltpu.sync_copy(dinvb, dinv_hbm.at[pl.ds(c * (NROW // 2) + s * TRN, TRN)])

    @pl.when(jnp.logical_and(c == 0, s == 0))
    def _():
        pltpu.sync_copy(cnt_sh, cnt_hbm)


def _prep(dst_p, batch2d, iota8, zeros3344, ones128):
    return pl.kernel(
        _prep_body,
        out_type=(jax.ShapeDtypeStruct((NROW, 16), f32),
                  jax.ShapeDtypeStruct((GR, 16), f32)),
        mesh=_mesh(),
        compiler_params=_sc_params,
        scratch_types=[
            pltpu.VMEM((8, CH), i32),
            pltpu.SemaphoreType.DMA,
            pltpu.VMEM((CH, 16), f32),
            pltpu.VMEM((GR, 16), f32),
            pltpu.VMEM((NROW // NS, 16), i32),
            pltpu.VMEM((8, 128), i32),
            pltpu.VMEM((TPN, 16), f32),
            pltpu.VMEM((TRN, 16), f32),
            pltpu.VMEM_SHARED((DEGR, 16), f32),
            pltpu.VMEM_SHARED((GR, 16), f32),
        ],
    )(dst_p, batch2d, iota8, zeros3344, ones128)


KC = 2
NSUP = NCHUNK // KC


def _prop_body(u_hbm, src_hbm, dst_hbm, p_hbm, srcb, dstb, rowb, acc,
               gsem, ssem):
    c = lax.axis_index("c")
    s = lax.axis_index("s")
    node_base = c * HALF + s * TPN
    lo = c * HALF
    dump = HALF + s

    pltpu.sync_copy(u_hbm.at[pl.ds(node_base, TPN)],
                    acc.at[pl.ds(s * TPN, TPN)])
    plsc.subcore_barrier()

    def chunk(j, carry):
        base = s * (NCHUNK // KC) * KC + j * KC
        pltpu.sync_copy(src_hbm.at[pl.ds(base, KC)], srcb)
        gd = [pltpu.async_copy(u_hbm.at[srcb.at[k]], rowb.at[k], gsem)
              for k in range(KC)]
        pltpu.sync_copy(dst_hbm.at[pl.ds(base, KC)], dstb)
        for k in range(KC):
            for g in range(CH // 16):
                d = dstb[k, pl.ds(g * 16, 16)]
                ld = d - lo
                m = jnp.logical_and(ld >= 0, ld < HALF)
                dstb[k, pl.ds(g * 16, 16)] = jnp.where(m, ld, dump)
        sd = []
        for k in range(KC):
            gd[k].wait()
            sd.append(pltpu.async_copy(rowb.at[k], acc.at[dstb.at[k]],
                                       ssem, add=True))
        for d_ in sd:
            d_.wait()
        return carry
    lax.fori_loop(0, NSUP, chunk, 0)

    plsc.subcore_barrier()
    pltpu.sync_copy(acc.at[pl.ds(s * TPN, TPN)],
                    p_hbm.at[pl.ds(node_base, TPN)])


def _prop(u, src2, dst2):
    return pl.kernel(
        _prop_body,
        out_type=jax.ShapeDtypeStruct((NP, H), f32),
        mesh=_mesh(),
        compiler_params=pltpu.CompilerParams(use_tc_tiling_on_sc=False),
        scratch_types=[
            pltpu.VMEM((KC, CH), i32),
            pltpu.VMEM((KC, CH), i32),
            pltpu.VMEM((KC, CH, H), f32),
            pltpu.VMEM_SHARED((ACCR, H), f32),
            pltpu.SemaphoreType.DMA,
            pltpu.SemaphoreType.DMA,
        ],
    )(u, src2, dst2)


def _pool_body(z_hbm, batch_hbm, iota_hbm, zeros_hbm, psum_hbm,
               zb, bb, pp, iv, psh):
    c = lax.axis_index("c")
    s = lax.axis_index("s")
    row_base = c * (NROW // 2) + s * TRN
    lane = jnp.arange(16, dtype=i32)

    pltpu.sync_copy(zeros_hbm.at[pl.ds(0, GR)], pp)
    pltpu.sync_copy(zeros_hbm.at[pl.ds(0, GR // NS)],
                    psh.at[pl.ds(s * (GR // NS), GR // NS)])
    pltpu.sync_copy(iota_hbm, iv)
    pltpu.sync_copy(z_hbm.at[pl.ds(row_base, TRN)], zb)
    pltpu.sync_copy(batch_hbm.at[pl.ds(row_base, TRN)], bb)
    plsc.subcore_barrier()
    for r in range(TRN):
        plsc.addupdate_scatter(pp, [bb[r], lane], zb[r])
    for k in range(GR // 128):
        pltpu.sync_copy(pp.at[pl.ds(k * 128, 128)],
                        psh.at[iv.at[k]], add=True)
    plsc.subcore_barrier()
    @pl.when(s == 0)
    def _():
        pltpu.sync_copy(psh, psum_hbm.at[c])


def _pool(z2d, batch2d, iota8, zeros3344):
    return pl.kernel(
        _pool_body,
        out_type=jax.ShapeDtypeStruct((NC, GR, 16), f32),
        mesh=_mesh(),
        compiler_params=_sc_params,
        scratch_types=[
            pltpu.VMEM((TRN, 16), f32),
            pltpu.VMEM((TRN, 16), i32),
            pltpu.VMEM((GR, 16), f32),
            pltpu.VMEM((8, 128), i32),
            pltpu.VMEM_SHARED((GR, 16), f32),
        ],
    )(z2d, batch2d, iota8, zeros3344)


def _lrelu(x):
    return jnp.where(x >= 0, x, 0.01 * x)


def _tc_in_body(fts_ref, w_ref, dinv_ref, o_ref):
    xw = jnp.dot(fts_ref[...], w_ref[...], preferred_element_type=f32)
    o_ref[...] = dinv_ref[...] * xw


def _tc_mid_body(p_ref, dinv_ref, b_ref, w_ref, o_ref):
    h = _lrelu(dinv_ref[...] * p_ref[...] + b_ref[...])
    o_ref[...] = dinv_ref[...] * jnp.dot(h, w_ref[...],
                                         preferred_element_type=f32)


def _tc_out_body(p_ref, dinv_ref, b3_ref, wl_ref, bl_ref, wf_ref, o_ref):
    h3 = dinv_ref[...] * p_ref[...] + b3_ref[...]
    t = _lrelu(jnp.dot(h3, wl_ref[...], preferred_element_type=f32)
               + bl_ref[...])
    o_ref[...] = jnp.sum(t * wf_ref[...], axis=1, keepdims=True)


def _tc_fin_body(ps_ref, cnt_ref, bf_ref, o_ref):
    st = ps_ref[:GR, :] + ps_ref[GR:, :]
    pooled = jnp.sum(st, axis=1, keepdims=True)
    counts = jnp.sum(cnt_ref[...], axis=1, keepdims=True)
    mean = pooled[:G] / jnp.maximum(counts[:G], 1.0)
    o_ref[...] = mean + bf_ref[0]


_R = 512
_GRID = NP // _R


def _row_spec(w):
    return pl.BlockSpec((_R, w), lambda i: (i, 0))


def _full_spec(a, b):
    return pl.BlockSpec((a, b), lambda i: (0, 0))


def _vec_spec(n):
    return pl.BlockSpec((n,), lambda i: (0,))


def _tc_in(fts_p, w1p, dinv2):
    return pl.pallas_call(
        _tc_in_body,
        grid=(_GRID,),
        in_specs=[_row_spec(16), _full_spec(16, H), _row_spec(1)],
        out_specs=_row_spec(H),
        out_shape=jax.ShapeDtypeStruct((NP, H), f32),
    )(fts_p, w1p, dinv2)


def _tc_mid(p, dinv2, b, w):
    return pl.pallas_call(
        _tc_mid_body,
        grid=(_GRID,),
        in_specs=[_row_spec(H), _row_spec(1), _vec_spec(H),
                  _full_spec(H, H)],
        out_specs=_row_spec(H),
        out_shape=jax.ShapeDtypeStruct((NP, H), f32),
    )(p, dinv2, b, w)


def _tc_out(p, dinv2, b3, wl, bl, wfr):
    return pl.pallas_call(
        _tc_out_body,
        grid=(_GRID,),
        in_specs=[_row_spec(H), _row_spec(1), _vec_spec(H),
                  _full_spec(H, H), _vec_spec(H), _vec_spec(H)],
        out_specs=_row_spec(1),
        out_shape=jax.ShapeDtypeStruct((NP, 1), f32),
    )(p, dinv2, b3, wl, bl, wfr)


def _tc_fin(psumf, cntf, bf):
    return pl.pallas_call(
        _tc_fin_body,
        grid=(1,),
        in_specs=[_full_spec(NC * GR, 16), _full_spec(GR, 16),
                  _vec_spec(1)],
        out_specs=pl.BlockSpec((G, 1), lambda i: (0, 0)),
        out_shape=jax.ShapeDtypeStruct((G, 1), f32),
    )(psumf, cntf, bf)


def kernel(fts, adj, batch, W1, b1, W2, b2, W3, b3, Wl, bl, Wf, bf):
    fts_p = jnp.zeros((NP, 16), f32).at[:N, :IN].set(fts)
    w1p = jnp.zeros((16, H), f32).at[:IN, :].set(W1)
    src_p = jnp.zeros((EP,), i32).at[:E].set(adj[0]).reshape(EP // CH, CH)
    dst_p = jnp.full((EP,), DUMP_DST, i32).at[:E].set(adj[1]).reshape(
        EP // CH, CH)
    batch_p = jnp.full((NP,), G, i32).at[:N].set(batch)
    batch2d = batch_p.reshape(NROW, 16)
    iota8 = jnp.arange(8 * 128, dtype=i32).reshape(8, 128)
    zeros3344 = jnp.zeros((DEGR // NS, 16), f32)
    ones128 = jnp.ones((CH, 16), f32)

    dinv2d, cnt2d = _prep(dst_p, batch2d, iota8, zeros3344, ones128)
    dinv2 = dinv2d.reshape(NP, 1)

    u1 = _tc_in(fts_p, w1p, dinv2)
    p1 = _prop(u1, src_p, dst_p)
    u2 = _tc_mid(p1, dinv2, b1, W2)
    p2 = _prop(u2, src_p, dst_p)
    u3 = _tc_mid(p2, dinv2, b2, W3)
    p3 = _prop(u3, src_p, dst_p)
    z = _tc_out(p3, dinv2, b3, Wl, bl, Wf.reshape(H))
    psum = _pool(z.reshape(NROW, 16), batch2d, iota8, zeros3344)
    out = _tc_fin(psum.reshape(NC * GR, 16), cnt2d, bf)
    return out


# speedup vs baseline: 8.6073x; 1.0263x over previous
import functools

import jax
import jax.numpy as jnp
from jax import lax
from jax.experimental import pallas as pl
from jax.experimental.pallas import tpu as pltpu
from jax.experimental.pallas import tpu_sc as plsc

N = 50000
E = 800000
IN = 9
H = 64
G = 512

NC = 2
NS = 16

NP = 53248
NROW = NP // 16
HALF = NP // 2
TPN = HALF // NS
TRN = TPN // 16
EP = 819200
EPT = EP // NS
CH = 128
NCHUNK = EPT // CH
DEGR = 53504
DUMP_DST = 53500
ACCR = HALF + NS
GR = 640
MAGIC = 0x5F3759DF

_mesh = functools.partial(
    plsc.VectorSubcoreMesh, core_axis_name="c", subcore_axis_name="s",
    num_cores=NC, num_subcores=NS)
_sc_params = pltpu.CompilerParams(
    needs_layout_passes=False, use_tc_tiling_on_sc=False)
f32 = jnp.float32
i32 = jnp.int32


def _prep_body(dst_hbm, batch_hbm, iota_hbm, zeros_hbm, ones_hbm,
               dinv_hbm, cnt_hbm,
               ebuf, dsem, onesb, cntp, bbuf, iv, draw, dinvb, deg_sh,
               cnt_sh):
    c = lax.axis_index("c")
    s = lax.axis_index("s")
    ones = jnp.full((16,), 1.0, f32)
    lane = jnp.arange(16, dtype=i32)
    zeros16 = jnp.zeros((16,), i32)

    pltpu.sync_copy(zeros_hbm.at[pl.ds(0, DEGR // NS)],
                    deg_sh.at[pl.ds(s * (DEGR // NS), DEGR // NS)])
    pltpu.sync_copy(zeros_hbm.at[pl.ds(0, GR // NS)],
                    cnt_sh.at[pl.ds(s * (GR // NS), GR // NS)])
    pltpu.sync_copy(zeros_hbm.at[pl.ds(0, GR)], cntp)
    pltpu.sync_copy(ones_hbm, onesb)
    pltpu.sync_copy(iota_hbm, iv)
    plsc.subcore_barrier()

    def deg_chunk(j, carry):
        base = s * (NCHUNK // 8) * 8 + j * 8
        pltpu.sync_copy(dst_hbm.at[pl.ds(base, 8)], ebuf)
        dd = [pltpu.async_copy(onesb, deg_sh.at[ebuf.at[k]], dsem, add=True)
              for k in range(8)]
        for d_ in dd:
            d_.wait()
        return carry
    lax.fori_loop(0, NCHUNK // 8, deg_chunk, 0)

    @pl.when(c == 0)
    def _():
        pltpu.sync_copy(batch_hbm.at[pl.ds(s * (NROW // NS), NROW // NS)],
                        bbuf)
        for r in range(NROW // NS):
            plsc.addupdate_scatter(cntp, [bbuf[r], lane], ones)

    plsc.subcore_barrier()

    @pl.when(c == 0)
    def _():
        for k in range(GR // 128):
            pltpu.sync_copy(cntp.at[pl.ds(k * 128, 128)],
                            cnt_sh.at[iv.at[k]], add=True)

    plsc.subcore_barrier()

    node_base = c * HALF + s * TPN
    pltpu.sync_copy(deg_sh.at[pl.ds(node_base, TPN)], draw)
    for r in range(TRN):
        x = plsc.load_gather(draw, [r * 16 + lane, zeros16]) + 1.0
        xi = plsc.bitcast(x, i32)
        y = plsc.bitcast(MAGIC - lax.shift_right_logical(xi, 1), f32)
        for _ in range(3):
            y = y * (1.5 - 0.5 * x * y * y)
        dinvb[r] = y
    pltpu.sync_copy(dinvb, dinv_hbm.at[pl.ds(c * (NROW // 2) + s * TRN, TRN)])

    @pl.when(jnp.logical_and(c == 0, s == 0))
    def _():
        pltpu.sync_copy(cnt_sh, cnt_hbm)


def _prep(dst_p, batch2d, iota8, zeros3344, ones128):
    return pl.kernel(
        _prep_body,
        out_type=(jax.ShapeDtypeStruct((NROW, 16), f32),
                  jax.ShapeDtypeStruct((GR, 16), f32)),
        mesh=_mesh(),
        compiler_params=_sc_params,
        scratch_types=[
            pltpu.VMEM((8, CH), i32),
            pltpu.SemaphoreType.DMA,
            pltpu.VMEM((CH, 16), f32),
            pltpu.VMEM((GR, 16), f32),
            pltpu.VMEM((NROW // NS, 16), i32),
            pltpu.VMEM((8, 128), i32),
            pltpu.VMEM((TPN, 16), f32),
            pltpu.VMEM((TRN, 16), f32),
            pltpu.VMEM_SHARED((DEGR, 16), f32),
            pltpu.VMEM_SHARED((GR, 16), f32),
        ],
    )(dst_p, batch2d, iota8, zeros3344, ones128)


SB = 4
NSTRIP = NCHUNK // SB


def _prop_body(u_hbm, src_hbm, dst_hbm, p_hbm, srcb, dstb, dumpb, rowb, acc,
               gsem, ssem):
    c = lax.axis_index("c")
    s = lax.axis_index("s")
    node_base = c * HALF + s * TPN
    lo = c * HALF
    dump = HALF + s

    pltpu.sync_copy(u_hbm.at[pl.ds(node_base, TPN)],
                    acc.at[pl.ds(s * TPN, TPN)])

    for g in range(CH // 16):
        dumpb[0, pl.ds(g * 16, 16)] = jnp.zeros((16,), i32) + dump
    plsc.subcore_barrier()
    for b in range(2):
        pltpu.async_copy(rowb.at[b], acc.at[dumpb.at[0]], ssem, add=True)

    def _drain_one(b):
        pltpu.make_async_copy(u_hbm.at[pl.ds(0, CH)], rowb.at[b], ssem).wait()

    def strip2(j, carry):
        for jj in range(2):
            si = 2 * j + jj
            base = s * NCHUNK + si * SB
            pltpu.sync_copy(src_hbm.at[pl.ds(base, SB)], srcb.at[jj])
            pltpu.sync_copy(dst_hbm.at[pl.ds(base, SB)], dstb.at[jj])
            for k in range(SB):
                for g in range(CH // 16):
                    d = dstb[jj, k, pl.ds(g * 16, 16)]
                    ld = d - lo
                    m = jnp.logical_and(ld >= 0, ld < HALF)
                    dstb[jj, k, pl.ds(g * 16, 16)] = jnp.where(m, ld, dump)
            gd = [None, None]
            for k in range(SB):
                b = k % 2
                _drain_one(b)
                gd[b] = pltpu.async_copy(u_hbm.at[srcb.at[jj, k]],
                                         rowb.at[b], gsem)
                if k >= 1:
                    gd[1 - b].wait()
                    pltpu.async_copy(rowb.at[1 - b],
                                     acc.at[dstb.at[jj, k - 1]],
                                     ssem, add=True)
            gd[(SB - 1) % 2].wait()
            pltpu.async_copy(rowb.at[(SB - 1) % 2],
                             acc.at[dstb.at[jj, SB - 1]], ssem, add=True)
        return carry
    lax.fori_loop(0, NSTRIP // 2, strip2, 0)

    _drain_one(0)
    _drain_one(1)
    plsc.subcore_barrier()
    pltpu.sync_copy(acc.at[pl.ds(s * TPN, TPN)],
                    p_hbm.at[pl.ds(node_base, TPN)])


def _prop(u, src2, dst2):
    return pl.kernel(
        _prop_body,
        out_type=jax.ShapeDtypeStruct((NP, H), f32),
        mesh=_mesh(),
        compiler_params=pltpu.CompilerParams(use_tc_tiling_on_sc=False),
        scratch_types=[
            pltpu.VMEM((2, SB, CH), i32),
            pltpu.VMEM((2, SB, CH), i32),
            pltpu.VMEM((1, CH), i32),
            pltpu.VMEM((2, CH, H), f32),
            pltpu.VMEM_SHARED((ACCR, H), f32),
            pltpu.SemaphoreType.DMA,
            pltpu.SemaphoreType.DMA,
        ],
    )(u, src2, dst2)


def _pool_body(z_hbm, batch_hbm, iota_hbm, zeros_hbm, psum_hbm,
               zb, bb, pp, iv, psh):
    c = lax.axis_index("c")
    s = lax.axis_index("s")
    row_base = c * (NROW // 2) + s * TRN
    lane = jnp.arange(16, dtype=i32)

    pltpu.sync_copy(zeros_hbm.at[pl.ds(0, GR)], pp)
    pltpu.sync_copy(zeros_hbm.at[pl.ds(0, GR // NS)],
                    psh.at[pl.ds(s * (GR // NS), GR // NS)])
    pltpu.sync_copy(iota_hbm, iv)
    pltpu.sync_copy(z_hbm.at[pl.ds(row_base, TRN)], zb)
    pltpu.sync_copy(batch_hbm.at[pl.ds(row_base, TRN)], bb)
    plsc.subcore_barrier()
    for r in range(TRN):
        plsc.addupdate_scatter(pp, [bb[r], lane], zb[r])
    for k in range(GR // 128):
        pltpu.sync_copy(pp.at[pl.ds(k * 128, 128)],
                        psh.at[iv.at[k]], add=True)
    plsc.subcore_barrier()
    @pl.when(s == 0)
    def _():
        pltpu.sync_copy(psh, psum_hbm.at[c])


def _pool(z2d, batch2d, iota8, zeros3344):
    return pl.kernel(
        _pool_body,
        out_type=jax.ShapeDtypeStruct((NC, GR, 16), f32),
        mesh=_mesh(),
        compiler_params=_sc_params,
        scratch_types=[
            pltpu.VMEM((TRN, 16), f32),
            pltpu.VMEM((TRN, 16), i32),
            pltpu.VMEM((GR, 16), f32),
            pltpu.VMEM((8, 128), i32),
            pltpu.VMEM_SHARED((GR, 16), f32),
        ],
    )(z2d, batch2d, iota8, zeros3344)


def _lrelu(x):
    return jnp.where(x >= 0, x, 0.01 * x)


def _tc_in_body(fts_ref, w_ref, dinv_ref, o_ref):
    xw = jnp.dot(fts_ref[...], w_ref[...], preferred_element_type=f32)
    o_ref[...] = dinv_ref[...] * xw


def _tc_mid_body(p_ref, dinv_ref, b_ref, w_ref, o_ref):
    h = _lrelu(dinv_ref[...] * p_ref[...] + b_ref[...])
    o_ref[...] = dinv_ref[...] * jnp.dot(h, w_ref[...],
                                         preferred_element_type=f32)


def _tc_out_body(p_ref, dinv_ref, b3_ref, wl_ref, bl_ref, wf_ref, o_ref):
    h3 = dinv_ref[...] * p_ref[...] + b3_ref[...]
    t = _lrelu(jnp.dot(h3, wl_ref[...], preferred_element_type=f32)
               + bl_ref[...])
    o_ref[...] = jnp.sum(t * wf_ref[...], axis=1, keepdims=True)


def _tc_fin_body(ps_ref, cnt_ref, bf_ref, o_ref):
    st = ps_ref[:GR, :] + ps_ref[GR:, :]
    pooled = jnp.sum(st, axis=1, keepdims=True)
    counts = jnp.sum(cnt_ref[...], axis=1, keepdims=True)
    mean = pooled[:G] / jnp.maximum(counts[:G], 1.0)
    o_ref[...] = mean + bf_ref[0]


_R = 512
_GRID = NP // _R


def _row_spec(w):
    return pl.BlockSpec((_R, w), lambda i: (i, 0))


def _full_spec(a, b):
    return pl.BlockSpec((a, b), lambda i: (0, 0))


def _vec_spec(n):
    return pl.BlockSpec((n,), lambda i: (0,))


def _tc_in(fts_p, w1p, dinv2):
    return pl.pallas_call(
        _tc_in_body,
        grid=(_GRID,),
        in_specs=[_row_spec(16), _full_spec(16, H), _row_spec(1)],
        out_specs=_row_spec(H),
        out_shape=jax.ShapeDtypeStruct((NP, H), f32),
    )(fts_p, w1p, dinv2)


def _tc_mid(p, dinv2, b, w):
    return pl.pallas_call(
        _tc_mid_body,
        grid=(_GRID,),
        in_specs=[_row_spec(H), _row_spec(1), _vec_spec(H),
                  _full_spec(H, H)],
        out_specs=_row_spec(H),
        out_shape=jax.ShapeDtypeStruct((NP, H), f32),
    )(p, dinv2, b, w)


def _tc_out(p, dinv2, b3, wl, bl, wfr):
    return pl.pallas_call(
        _tc_out_body,
        grid=(_GRID,),
        in_specs=[_row_spec(H), _row_spec(1), _vec_spec(H),
                  _full_spec(H, H), _vec_spec(H), _vec_spec(H)],
        out_specs=_row_spec(1),
        out_shape=jax.ShapeDtypeStruct((NP, 1), f32),
    )(p, dinv2, b3, wl, bl, wfr)


def _tc_fin(psumf, cntf, bf):
    return pl.pallas_call(
        _tc_fin_body,
        grid=(1,),
        in_specs=[_full_spec(NC * GR, 16), _full_spec(GR, 16),
                  _vec_spec(1)],
        out_specs=pl.BlockSpec((G, 1), lambda i: (0, 0)),
        out_shape=jax.ShapeDtypeStruct((G, 1), f32),
    )(psumf, cntf, bf)


def kernel(fts, adj, batch, W1, b1, W2, b2, W3, b3, Wl, bl, Wf, bf):
    fts_p = jnp.zeros((NP, 16), f32).at[:N, :IN].set(fts)
    w1p = jnp.zeros((16, H), f32).at[:IN, :].set(W1)
    src_p = jnp.zeros((EP,), i32).at[:E].set(adj[0]).reshape(EP // CH, CH)
    dst_p = jnp.full((EP,), DUMP_DST, i32).at[:E].set(adj[1]).reshape(
        EP // CH, CH)
    batch_p = jnp.full((NP,), G, i32).at[:N].set(batch)
    batch2d = batch_p.reshape(NROW, 16)
    iota8 = jnp.arange(8 * 128, dtype=i32).reshape(8, 128)
    zeros3344 = jnp.zeros((DEGR // NS, 16), f32)
    ones128 = jnp.ones((CH, 16), f32)

    dinv2d, cnt2d = _prep(dst_p, batch2d, iota8, zeros3344, ones128)
    dinv2 = dinv2d.reshape(NP, 1)

    u1 = _tc_in(fts_p, w1p, dinv2)
    p1 = _prop(u1, src_p, dst_p)
    u2 = _tc_mid(p1, dinv2, b1, W2)
    p2 = _prop(u2, src_p, dst_p)
    u3 = _tc_mid(p2, dinv2, b2, W3)
    p3 = _prop(u3, src_p, dst_p)
    z = _tc_out(p3, dinv2, b3, Wl, bl, Wf.reshape(H))
    psum = _pool(z.reshape(NROW, 16), batch2d, iota8, zeros3344)
    out = _tc_fin(psum.reshape(NC * GR, 16), cnt2d, bf)
    return out
